# baseline (speedup 1.0000x reference)
import functools

import jax
import jax.numpy as jnp
from jax import lax
from jax.experimental import pallas as pl
from jax.experimental.pallas import tpu as pltpu
from jax.experimental.pallas import tpu_sc as plsc

_NC = 2
_NS = 16
_NW = _NC * _NS

_CW = 64


def _pool_sc(q_idx4, c_idx4, q_table, c_table, *, B, L, D):
    rows_per_w = B // _NW
    nk = rows_per_w // _CW

    mesh = plsc.VectorSubcoreMesh(core_axis_name="c", subcore_axis_name="s")

    @functools.partial(
        pl.kernel,
        mesh=mesh,
        out_type=(
            jax.ShapeDtypeStruct((B, D), jnp.float32),
            jax.ShapeDtypeStruct((B, D), jnp.float32),
        ),
        scratch_types=[
            pltpu.VMEM((L, nk, _CW), jnp.int32),
            pltpu.VMEM((L, nk, _CW), jnp.int32),
            pltpu.VMEM((rows_per_w, D), jnp.float32),
            [pltpu.SemaphoreType.DMA] * nk,
            [pltpu.SemaphoreType.DMA] * nk,
        ],
    )
    def pool(q_idx_hbm, c_idx_hbm, q_tab_hbm, c_tab_hbm,
             q_out_hbm, c_out_hbm, idxq_v, idxc_v, acc_v, gsem, osem):
        wid = lax.axis_index("s") * _NC + lax.axis_index("c")
        base = wid * rows_per_w

        def win(k):
            return acc_v.at[pl.ds(k * _CW, _CW)]

        def fire_tower(idx_v, tab_hbm, inits):
            for k in range(nk):
                inits[k].wait()

                def passes(j, carry, k=k):
                    pltpu.async_copy(
                        tab_hbm.at[idx_v.at[j, k]], win(k), gsem[k],
                        add=True,
                    )
                    return carry

                lax.fori_loop(1, L, passes, 0)

        def drain_tower(tab_hbm, out_hbm):
            for k in range(nk):
                def drain(j, carry, k=k):
                    pltpu.make_async_copy(
                        tab_hbm.at[pl.ds(0, _CW)], win(k), gsem[k],
                    ).wait()
                    return carry

                lax.fori_loop(1, L, drain, 0)
                pltpu.async_copy(
                    win(k), out_hbm.at[pl.ds(base + k * _CW, _CW)], osem[k])

        pltpu.sync_copy(q_idx_hbm.at[wid], idxq_v)
        q_inits = [
            pltpu.async_copy(q_tab_hbm.at[idxq_v.at[0, k]], win(k), gsem[k])
            for k in range(nk)
        ]
        fire_tower(idxq_v, q_tab_hbm, q_inits)

        pltpu.sync_copy(c_idx_hbm.at[wid], idxc_v)

        drain_tower(q_tab_hbm, q_out_hbm)

        c_inits = []
        for k in range(nk):
            pltpu.make_async_copy(
                win(k), q_out_hbm.at[pl.ds(base + k * _CW, _CW)], osem[k],
            ).wait()
            c_inits.append(
                pltpu.async_copy(c_tab_hbm.at[idxc_v.at[0, k]], win(k),
                                 gsem[k]))
        fire_tower(idxc_v, c_tab_hbm, c_inits)
        drain_tower(c_tab_hbm, c_out_hbm)
        for k in range(nk):
            pltpu.make_async_copy(
                win(k), c_out_hbm.at[pl.ds(base + k * _CW, _CW)], osem[k],
            ).wait()

    return pool(q_idx4, c_idx4, q_table, c_table)


def _mlp_tc(q_pooled, c_pooled, q_ws, q_bs, c_ws, c_bs, *, B, D):
    blk = 8192
    grid = (B // blk,)

    n_layers = len(q_ws)
    out_d = q_ws[-1].shape[0]

    def body(qp_ref, cp_ref, *refs):
        q_wrefs = refs[0:n_layers]
        q_brefs = refs[n_layers:2 * n_layers]
        c_wrefs = refs[2 * n_layers:3 * n_layers]
        c_brefs = refs[3 * n_layers:4 * n_layers]
        q_out_ref, c_out_ref = refs[4 * n_layers:]

        def tower(x, wrefs, brefs):
            for w_ref, b_ref in zip(wrefs[:-1], brefs[:-1]):
                y = lax.dot_general(
                    x, w_ref[...], (((1,), (1,)), ((), ())),
                    preferred_element_type=jnp.float32,
                )
                x = jnp.maximum(y + b_ref[...], 0.0)
            y = lax.dot_general(
                wrefs[-1][...], x, (((1,), (1,)), ((), ())),
                preferred_element_type=jnp.float32,
            )
            return jnp.maximum(y + brefs[-1][...], 0.0)

        q_out_ref[...] = tower(qp_ref[...], q_wrefs, q_brefs)
        c_out_ref[...] = tower(cp_ref[...], c_wrefs, c_brefs)

    x_spec = pl.BlockSpec((blk, D), lambda i: (i, 0))
    full = lambda a: pl.BlockSpec(a.shape, lambda i: (0,) * a.ndim)
    in_specs = (
        [x_spec, x_spec]
        + [full(w) for w in q_ws] + [full(b) for b in q_bs]
        + [full(w) for w in c_ws] + [full(b) for b in c_bs]
    )
    out_specs = (
        pl.BlockSpec((out_d, blk), lambda i: (0, i)),
        pl.BlockSpec((out_d, blk), lambda i: (0, i)),
    )
    q_t, c_t = pl.pallas_call(
        body,
        grid=grid,
        in_specs=in_specs,
        out_specs=out_specs,
        out_shape=(
            jax.ShapeDtypeStruct((out_d, B), jnp.float32),
            jax.ShapeDtypeStruct((out_d, B), jnp.float32),
        ),
    )(q_pooled, c_pooled, *q_ws, *q_bs, *c_ws, *c_bs)
    return q_t.T, c_t.T


def kernel(query_indices, candidate_indices, q_table, c_table,
           q_w0, q_b0, q_w1, q_b1, q_w2, q_b2,
           c_w0, c_b0, c_w1, c_b1, c_w2, c_b2):
    B, L = query_indices.shape
    V, D = q_table.shape
    rows_per_w = B // _NW
    nk = rows_per_w // _CW

    def prep(idx):
        idx = idx.astype(jnp.int32)
        return idx.reshape(_NW, nk, _CW, L).transpose(0, 3, 1, 2)

    q_pooled, c_pooled = _pool_sc(
        prep(query_indices), prep(candidate_indices), q_table, c_table,
        B=B, L=L, D=D,
    )

    q_bs = [q_b0.reshape(1, -1), q_b1.reshape(1, -1), q_b2.reshape(-1, 1)]
    c_bs = [c_b0.reshape(1, -1), c_b1.reshape(1, -1), c_b2.reshape(-1, 1)]
    return _mlp_tc(
        q_pooled, c_pooled,
        [q_w0, q_w1, q_w2], q_bs, [c_w0, c_w1, c_w2], c_bs,
        B=B, D=D,
    )


# speedup vs baseline: 1.0349x; 1.0349x over previous
import functools

import jax
import jax.numpy as jnp
from jax import lax
from jax.experimental import pallas as pl
from jax.experimental.pallas import tpu as pltpu
from jax.experimental.pallas import tpu_sc as plsc

_NC = 2
_NS = 16
_NW = _NC * _NS

_CW = 128


def _pool_sc(q_idx4, c_idx4, q_table, c_table, *, B, L, D):
    rows_per_w = B // _NW
    nk = rows_per_w // _CW

    mesh = plsc.VectorSubcoreMesh(core_axis_name="c", subcore_axis_name="s")

    @functools.partial(
        pl.kernel,
        mesh=mesh,
        out_type=(
            jax.ShapeDtypeStruct((B, D), jnp.float32),
            jax.ShapeDtypeStruct((B, D), jnp.float32),
        ),
        scratch_types=[
            pltpu.VMEM((L, nk, _CW), jnp.int32),
            pltpu.VMEM((L, nk, _CW), jnp.int32),
            pltpu.VMEM((rows_per_w, D), jnp.float32),
            [pltpu.SemaphoreType.DMA] * nk,
            [pltpu.SemaphoreType.DMA] * nk,
        ],
    )
    def pool(q_idx_hbm, c_idx_hbm, q_tab_hbm, c_tab_hbm,
             q_out_hbm, c_out_hbm, idxq_v, idxc_v, acc_v, gsem, osem):
        wid = lax.axis_index("s") * _NC + lax.axis_index("c")
        base = wid * rows_per_w

        def win(k):
            return acc_v.at[pl.ds(k * _CW, _CW)]

        def fire_tower(idx_v, tab_hbm, inits):
            for k in range(nk):
                inits[k].wait()

                def passes(j, carry, k=k):
                    pltpu.async_copy(
                        tab_hbm.at[idx_v.at[j, k]], win(k), gsem[k],
                        add=True,
                    )
                    return carry

                lax.fori_loop(1, L, passes, 0)

        def drain_tower(tab_hbm, out_hbm):
            for k in range(nk):
                def drain(j, carry, k=k):
                    pltpu.make_async_copy(
                        tab_hbm.at[pl.ds(0, _CW)], win(k), gsem[k],
                    ).wait()
                    return carry

                lax.fori_loop(1, L, drain, 0)
                pltpu.async_copy(
                    win(k), out_hbm.at[pl.ds(base + k * _CW, _CW)], osem[k])

        pltpu.sync_copy(q_idx_hbm.at[wid, 0], idxq_v.at[0])
        q_inits = [
            pltpu.async_copy(q_tab_hbm.at[idxq_v.at[0, k]], win(k), gsem[k])
            for k in range(nk)
        ]
        pltpu.sync_copy(q_idx_hbm.at[wid, pl.ds(1, L - 1)],
                        idxq_v.at[pl.ds(1, L - 1)])
        fire_tower(idxq_v, q_tab_hbm, q_inits)

        pltpu.sync_copy(c_idx_hbm.at[wid], idxc_v)

        drain_tower(q_tab_hbm, q_out_hbm)

        c_inits = []
        for k in range(nk):
            pltpu.make_async_copy(
                win(k), q_out_hbm.at[pl.ds(base + k * _CW, _CW)], osem[k],
            ).wait()
            c_inits.append(
                pltpu.async_copy(c_tab_hbm.at[idxc_v.at[0, k]], win(k),
                                 gsem[k]))
        fire_tower(idxc_v, c_tab_hbm, c_inits)
        drain_tower(c_tab_hbm, c_out_hbm)
        for k in range(nk):
            pltpu.make_async_copy(
                win(k), c_out_hbm.at[pl.ds(base + k * _CW, _CW)], osem[k],
            ).wait()

    return pool(q_idx4, c_idx4, q_table, c_table)


def _mlp_tc(q_pooled, c_pooled, q_ws, q_bs, c_ws, c_bs, *, B, D):
    blk = 8192
    grid = (B // blk,)

    n_layers = len(q_ws)
    out_d = q_ws[-1].shape[0]

    def body(qp_ref, cp_ref, *refs):
        q_wrefs = refs[0:n_layers]
        q_brefs = refs[n_layers:2 * n_layers]
        c_wrefs = refs[2 * n_layers:3 * n_layers]
        c_brefs = refs[3 * n_layers:4 * n_layers]
        q_out_ref, c_out_ref = refs[4 * n_layers:]

        def tower(x, wrefs, brefs):
            for w_ref, b_ref in zip(wrefs[:-1], brefs[:-1]):
                y = lax.dot_general(
                    x, w_ref[...], (((1,), (1,)), ((), ())),
                    preferred_element_type=jnp.float32,
                )
                x = jnp.maximum(y + b_ref[...], 0.0)
            y = lax.dot_general(
                wrefs[-1][...], x, (((1,), (1,)), ((), ())),
                preferred_element_type=jnp.float32,
            )
            return jnp.maximum(y + brefs[-1][...], 0.0)

        q_out_ref[...] = tower(qp_ref[...], q_wrefs, q_brefs)
        c_out_ref[...] = tower(cp_ref[...], c_wrefs, c_brefs)

    x_spec = pl.BlockSpec((blk, D), lambda i: (i, 0))
    full = lambda a: pl.BlockSpec(a.shape, lambda i: (0,) * a.ndim)
    in_specs = (
        [x_spec, x_spec]
        + [full(w) for w in q_ws] + [full(b) for b in q_bs]
        + [full(w) for w in c_ws] + [full(b) for b in c_bs]
    )
    out_specs = (
        pl.BlockSpec((out_d, blk), lambda i: (0, i)),
        pl.BlockSpec((out_d, blk), lambda i: (0, i)),
    )
    q_t, c_t = pl.pallas_call(
        body,
        grid=grid,
        in_specs=in_specs,
        out_specs=out_specs,
        out_shape=(
            jax.ShapeDtypeStruct((out_d, B), jnp.float32),
            jax.ShapeDtypeStruct((out_d, B), jnp.float32),
        ),
    )(q_pooled, c_pooled, *q_ws, *q_bs, *c_ws, *c_bs)
    return q_t.T, c_t.T


def kernel(query_indices, candidate_indices, q_table, c_table,
           q_w0, q_b0, q_w1, q_b1, q_w2, q_b2,
           c_w0, c_b0, c_w1, c_b1, c_w2, c_b2):
    B, L = query_indices.shape
    V, D = q_table.shape
    rows_per_w = B // _NW
    nk = rows_per_w // _CW

    def prep(idx):
        idx = idx.astype(jnp.int32)
        return idx.reshape(_NW, nk, _CW, L).transpose(0, 3, 1, 2)

    q_pooled, c_pooled = _pool_sc(
        prep(query_indices), prep(candidate_indices), q_table, c_table,
        B=B, L=L, D=D,
    )

    q_bs = [q_b0.reshape(1, -1), q_b1.reshape(1, -1), q_b2.reshape(-1, 1)]
    c_bs = [c_b0.reshape(1, -1), c_b1.reshape(1, -1), c_b2.reshape(-1, 1)]
    return _mlp_tc(
        q_pooled, c_pooled,
        [q_w0, q_w1, q_w2], q_bs, [c_w0, c_w1, c_w2], c_bs,
        B=B, D=D,
    )
